# baseline (speedup 1.0000x reference)
import functools

import numpy as np
import jax
import jax.numpy as jnp
from jax import lax
from jax.experimental import pallas as pl
from jax.experimental.pallas import tpu as pltpu
from jax.experimental.pallas import tpu_sc as plsc

_VOCAB = 100000
_SEQ = 2048
_D = 512
_B = 4
_NC = 2
_NS = 16
_NW = _NC * _NS
_PW = _SEQ // _NW
_VPR = _D // 16


def _pos_encoding():
    i = np.arange(_D // 2, dtype=np.float64)
    denom = np.power(10000.0, 2.0 * i / _D)
    pos = np.arange(_SEQ, dtype=np.float64)[:, None]
    pe = np.zeros((_SEQ, _D), dtype=np.float64)
    pe[:, 0::2] = np.sin(pos / denom)
    pe[:, 1::2] = np.cos(pos / denom)
    return jnp.asarray(pe, dtype=jnp.float32)


_mesh = plsc.VectorSubcoreMesh(core_axis_name="c", subcore_axis_name="s")


@functools.partial(
    pl.kernel,
    mesh=_mesh,
    out_type=jax.ShapeDtypeStruct((_B * _SEQ, _D), jnp.float32),
    scratch_types=[
        pltpu.VMEM((_B, _PW), jnp.int32),
        pltpu.VMEM((_PW, _D), jnp.float32),
        pltpu.VMEM((_PW, _D), jnp.float32),
        pltpu.SemaphoreType.DMA,
    ],
)
def _emb_kernel(idx_hbm, table_hbm, pos_hbm, out_hbm, idx_v, pos_v, row_v, sem):
    c = lax.axis_index("c")
    s = lax.axis_index("s")
    w = s * _NC + c
    p0 = w * _PW

    pltpu.sync_copy(idx_hbm.at[w], idx_v)
    pltpu.sync_copy(pos_hbm.at[pl.ds(p0, _PW)], pos_v)

    for b in range(_B):
        pltpu.async_copy(table_hbm.at[idx_v.at[b]], row_v, sem).wait()

        def add_row(r):
            for j in range(_VPR):
                v = pos_v[r, pl.ds(j * 16, 16)]
                plsc.addupdate(row_v.at[r, pl.ds(j * 16, 16)], v)

        pl.loop(0, _PW)(add_row)
        pltpu.sync_copy(row_v, out_hbm.at[pl.ds(b * _SEQ + p0, _PW)])


def kernel(inputs, table):
    idx = inputs.astype(jnp.int32).reshape(_B, _NW, _PW).transpose(1, 0, 2)
    out = _emb_kernel(idx, table, _pos_encoding())
    return out.reshape(_B, _SEQ, _D)


# speedup vs baseline: 1.1098x; 1.1098x over previous
import functools

import numpy as np
import jax
import jax.numpy as jnp
from jax import lax
from jax.experimental import pallas as pl
from jax.experimental.pallas import tpu as pltpu
from jax.experimental.pallas import tpu_sc as plsc

_VOCAB = 100000
_SEQ = 2048
_D = 512
_B = 4
_NC = 2
_NS = 16
_NW = _NC * _NS
_PW = _SEQ // _NW
_VPR = _D // 16


def _pos_encoding():
    i = np.arange(_D // 2, dtype=np.float64)
    denom = np.power(10000.0, 2.0 * i / _D)
    pos = np.arange(_SEQ, dtype=np.float64)[:, None]
    pe = np.zeros((_SEQ, _D), dtype=np.float64)
    pe[:, 0::2] = np.sin(pos / denom)
    pe[:, 1::2] = np.cos(pos / denom)
    return jnp.asarray(pe, dtype=jnp.float32)


_mesh = plsc.VectorSubcoreMesh(core_axis_name="c", subcore_axis_name="s")


@functools.partial(
    pl.kernel,
    mesh=_mesh,
    out_type=jax.ShapeDtypeStruct((_B * _SEQ, _D), jnp.float32),
    scratch_types=[
        pltpu.VMEM((_B, _PW), jnp.int32),
        pltpu.VMEM((_PW, _D), jnp.float32),
        pltpu.VMEM((_PW, _D), jnp.float32),
        pltpu.VMEM((_PW, _D), jnp.float32),
        pltpu.SemaphoreType.DMA,
        pltpu.SemaphoreType.DMA,
        pltpu.SemaphoreType.DMA,
        pltpu.SemaphoreType.DMA,
    ],
)
def _emb_kernel(idx_hbm, table_hbm, pos_hbm, out_hbm,
                idx_v, pos_v, rv0, rv1, gs0, gs1, os0, os1):
    c = lax.axis_index("c")
    s = lax.axis_index("s")
    w = s * _NC + c
    p0 = w * _PW

    rv = (rv0, rv1)
    gs = (gs0, gs1)
    osem = (os0, os1)

    pltpu.sync_copy(idx_hbm.at[w], idx_v)
    pltpu.sync_copy(pos_hbm.at[pl.ds(p0, _PW)], pos_v)

    def gather(b):
        return pltpu.async_copy(table_hbm.at[idx_v.at[b]], rv[b % 2], gs[b % 2])

    def out_copy(b):
        return pltpu.async_copy(
            rv[b % 2], out_hbm.at[pl.ds(b * _SEQ + p0, _PW)], osem[b % 2])

    def add_pos(row_ref):
        def body(r):
            for j in range(_VPR):
                v = pos_v[r, pl.ds(j * 16, 16)]
                plsc.addupdate(row_ref.at[r, pl.ds(j * 16, 16)], v)
        plsc.parallel_loop(0, _PW, unroll=2)(body)

    gd = [None] * _B
    od = [None] * _B
    gd[0] = gather(0)
    gd[1] = gather(1)
    for b in range(_B):
        gd[b].wait()
        add_pos(rv[b % 2])
        od[b] = out_copy(b)
        if b + 2 < _B:
            od[b].wait()
            gd[b + 2] = gather(b + 2)
    od[_B - 2].wait()
    od[_B - 1].wait()


def kernel(inputs, table):
    idx = inputs.astype(jnp.int32).reshape(_B, _NW, _PW).transpose(1, 0, 2)
    out = _emb_kernel(idx, table, _pos_encoding())
    return out.reshape(_B, _SEQ, _D)
